# Initial kernel scaffold; baseline (speedup 1.0000x reference)
import jax
import jax.numpy as jnp
from jax.experimental import pallas as pl


def kernel(x, edge_index, batch, params):
    raise NotImplementedError("write your pallas kernel here")


# speedup vs baseline: 4.2971x; 4.2971x over previous
import functools

import jax
import jax.numpy as jnp
from jax import lax
from jax.experimental import pallas as pl
from jax.experimental.pallas import tpu as pltpu
from jax.experimental.pallas import tpu_sc as plsc

_NC = 2
_NS = 16
_NW = _NC * _NS
_BN_EPS = 1e-5


@functools.partial(jax.jit, static_argnums=(3, 4, 5))
def _edge_agg(x, src_g, dst_g, n_pad, k, c):
    d = x.shape[1]
    rows_per_tile = n_pad // _NS
    zr = 64
    mesh = plsc.VectorSubcoreMesh(core_axis_name="c", subcore_axis_name="s")

    def body(x_hbm, src_hbm, dst_hbm, z_hbm, out_hbm, sidx, didx, buf, zbuf,
             acc, sem):
        ci = lax.axis_index("c")
        si = lax.axis_index("s")
        wid = si * _NC + ci

        pltpu.sync_copy(z_hbm, zbuf)

        def zacc(t, carry):
            pltpu.sync_copy(zbuf, acc.at[pl.ds(si * rows_per_tile + t * zr, zr)])
            return carry

        lax.fori_loop(0, rows_per_tile // zr, zacc, 0, unroll=False)

        pltpu.sync_copy(src_hbm.at[wid], sidx)
        pltpu.sync_copy(dst_hbm.at[wid], didx)
        plsc.subcore_barrier()

        def chunk(kk, carry):
            pltpu.async_copy(x_hbm.at[sidx.at[kk]], buf, sem).wait()
            pltpu.sync_copy(buf, acc.at[didx.at[kk]], add=True)
            return carry

        lax.fori_loop(0, k, chunk, 0, unroll=False)
        plsc.subcore_barrier()

        def wout(t, carry):
            base = si * rows_per_tile + t * c
            pltpu.sync_copy(acc.at[pl.ds(base, c)], buf)
            pltpu.sync_copy(buf, out_hbm.at[ci, pl.ds(base, c)])
            return carry

        lax.fori_loop(0, rows_per_tile // c, wout, 0, unroll=False)

    zeros = jnp.zeros((zr, d), jnp.float32)
    call = pl.kernel(
        body,
        out_type=jax.ShapeDtypeStruct((_NC, n_pad, d), jnp.float32),
        mesh=mesh,
        scratch_types=[
            pltpu.VMEM((k, c), jnp.int32),
            pltpu.VMEM((k, c), jnp.int32),
            pltpu.VMEM((c, d), jnp.float32),
            pltpu.VMEM((zr, d), jnp.float32),
            pltpu.VMEM_SHARED((n_pad, d), jnp.float32),
            pltpu.SemaphoreType.DMA,
        ],
    )
    return call(x, src_g, dst_g, zeros)


def _bn(h, g, b):
    m = jnp.mean(h, axis=0, keepdims=True)
    hc = h - m
    v = jnp.mean(hc * hc, axis=0, keepdims=True)
    return g * hc * lax.rsqrt(v + _BN_EPS) + b


def _mm(a, b):
    return lax.dot_general(a, b, (((1,), (0,)), ((), ())),
                           preferred_element_type=jnp.float32)


def _ohmm(oh, m):
    ohb = oh.astype(jnp.bfloat16)

    def dg(v):
        return lax.dot_general(ohb, v, (((1,), (0,)), ((), ())),
                               preferred_element_type=jnp.float32)

    m1 = m.astype(jnp.bfloat16)
    r = m - m1.astype(jnp.float32)
    m2 = r.astype(jnp.bfloat16)
    m3 = (r - m2.astype(jnp.float32)).astype(jnp.bfloat16)
    return dg(m1) + (dg(m2) + dg(m3))


def _conv_mlp(h, wa, ba, g1, b1, wb, bb):
    h = _mm(h, wa) + ba
    h = _bn(h, g1, b1)
    h = jnp.maximum(h, 0.0)
    return _mm(h, wb) + bb


def _tc1_body(n, x_ref, agg_ref, wa, ba, g1, b1, wb, bb, bg, bb2, vemb, out):
    a = agg_ref[0, pl.ds(0, n), :] + agg_ref[1, pl.ds(0, n), :]
    h = x_ref[...] + a
    h = _conv_mlp(h, wa[...], ba[...], g1[...], b1[...], wb[...], bb[...])
    h = _bn(h, bg[...], bb2[...])
    h = jnp.maximum(h, 0.0)
    out[...] = h + vemb[...]


def _tc2_body(n, g, y_ref, agg_ref, wa, ba, g1, b1, wb, bb, bg, bb2,
              mw1, mb1, mg1, mbe1, mw2, mb2, mg2, mbe2, vemb,
              bcol, brow, out):
    a = agg_ref[0, pl.ds(0, n), :] + agg_ref[1, pl.ds(0, n), :]
    h = y_ref[...] + a
    h = _conv_mlp(h, wa[...], ba[...], g1[...], b1[...], wb[...], bb[...])
    h = _bn(h, bg[...], bb2[...])
    post = jnp.maximum(h, 0.0)

    oh = (bcol[...] == lax.broadcasted_iota(jnp.int32, (n, g), 1)
          ).astype(jnp.float32)
    oht = (brow[...] == lax.broadcasted_iota(jnp.int32, (g, n), 0)
           ).astype(jnp.float32)

    pooled = _ohmm(oht, post)
    v0 = jnp.broadcast_to(vemb[...], pooled.shape)
    hv = _mm(pooled + v0, mw1[...]) + mb1[...]
    hv = _bn(hv, mg1[...], mbe1[...])
    hv = jnp.maximum(hv, 0.0)
    hv = _mm(hv, mw2[...]) + mb2[...]
    hv = _bn(hv, mg2[...], mbe2[...])
    v1 = jnp.maximum(hv, 0.0)

    out[...] = post + _ohmm(oh, v1)


def _tc3_body(n, g, y_ref, agg_ref, wa, ba, g1, b1, wb, bb, bg, bb2,
              brow, out):
    a = agg_ref[0, pl.ds(0, n), :] + agg_ref[1, pl.ds(0, n), :]
    h = y_ref[...] + a
    h = _conv_mlp(h, wa[...], ba[...], g1[...], b1[...], wb[...], bb[...])
    post = _bn(h, bg[...], bb2[...])

    oht = (brow[...] == lax.broadcasted_iota(jnp.int32, (g, n), 0)
           ).astype(jnp.float32)
    pooled = _ohmm(oht, post)
    counts = jnp.sum(oht, axis=1, keepdims=True)
    out[...] = pooled / jnp.maximum(counts, 1.0)


def _conv_args(p):
    f1 = p['Wa'].shape[1]
    f2 = p['Wb'].shape[1]
    return (p['Wa'], p['ba'].reshape(1, f1), p['g1'].reshape(1, f1),
            p['b1'].reshape(1, f1), p['Wb'], p['bb'].reshape(1, f2))


def _bn_args(p):
    f = p['g'].shape[0]
    return (p['g'].reshape(1, f), p['b'].reshape(1, f))


def kernel(x, edge_index, batch, params):
    n, d = x.shape
    e = edge_index.shape[1]
    g = 64

    c = 128
    k = -(-e // (_NW * c))
    e_pad = _NW * k * c
    n_pad = -(-n // (_NS * 64)) * (_NS * 64)

    src = jnp.concatenate(
        [edge_index[0], jnp.zeros((e_pad - e,), jnp.int32)]).reshape(_NW, k, c)
    dst = jnp.concatenate(
        [edge_index[1], jnp.full((e_pad - e,), n, jnp.int32)]).reshape(_NW, k, c)

    bcol = batch.reshape(n, 1)
    brow = batch.reshape(1, n)
    vemb = params['vemb'][0].reshape(1, d)
    mlp = params['vmlp']
    f1 = mlp['W1'].shape[1]
    f2 = mlp['W2'].shape[1]

    agg1 = _edge_agg(x, src, dst, n_pad, k, c)
    y1 = pl.pallas_call(
        functools.partial(_tc1_body, n),
        out_shape=jax.ShapeDtypeStruct((n, d), jnp.float32),
    )(x, agg1, *_conv_args(params['conv1']), *_bn_args(params['bn1']), vemb)

    agg2 = _edge_agg(y1, src, dst, n_pad, k, c)
    y2 = pl.pallas_call(
        functools.partial(_tc2_body, n, g),
        out_shape=jax.ShapeDtypeStruct((n, d), jnp.float32),
    )(y1, agg2, *_conv_args(params['convs'][0]), *_bn_args(params['bns'][0]),
      mlp['W1'], mlp['b1'].reshape(1, f1), mlp['g1'].reshape(1, f1),
      mlp['be1'].reshape(1, f1), mlp['W2'], mlp['b2'].reshape(1, f2),
      mlp['g2'].reshape(1, f2), mlp['be2'].reshape(1, f2), vemb, bcol, brow)

    agg3 = _edge_agg(y2, src, dst, n_pad, k, c)
    out = pl.pallas_call(
        functools.partial(_tc3_body, n, g),
        out_shape=jax.ShapeDtypeStruct((g, d), jnp.float32),
    )(y2, agg3, *_conv_args(params['convs'][1]), *_bn_args(params['bns'][1]),
      brow)
    return out
